# Initial kernel scaffold; baseline (speedup 1.0000x reference)
import jax
import jax.numpy as jnp
from jax.experimental import pallas as pl


def kernel(nfeats, edge_index, edge_weights, W1, b1, W2, b2, W3, b3, Wc, bc):
    raise NotImplementedError("write your pallas kernel here")


# speedup vs baseline: 6.3076x; 6.3076x over previous
import functools

import jax
import jax.numpy as jnp
import numpy as np
from jax import lax
from jax.experimental import pallas as pl
from jax.experimental.pallas import tpu as pltpu
from jax.experimental.pallas import tpu_sc as plsc

NC = 2
NS = 16
NW = NC * NS
LANES = 16
B = 128
ZR = 64


def _sc_mesh():
    return plsc.VectorSubcoreMesh(
        core_axis_name="c", subcore_axis_name="s", num_cores=NC, num_subcores=NS
    )


def _make_deg_nw(CH, NPAD):
    PT = NPAD // NS

    def body(src_hbm, w_hbm, nw_hbm, idx_v, w_v, deg_v, nw_v, zero_v, deg_sh, sem):
        c = lax.axis_index("c")
        s = lax.axis_index("s")
        wid = s * NC + c
        z16 = jnp.zeros((LANES,), jnp.float32)
        for g in range(PT // LANES):
            zero_v[pl.ds(g * LANES, LANES)] = z16
        pltpu.sync_copy(zero_v, deg_sh.at[pl.ds(s * PT, PT)])
        plsc.subcore_barrier()
        for half in range(2):
            slab = s * 2 + half
            pltpu.sync_copy(src_hbm.at[slab], idx_v)
            pltpu.sync_copy(w_hbm.at[slab], w_v)

            def acc_body(j, _):
                pltpu.sync_copy(w_v.at[j], deg_sh.at[idx_v.at[j]], add=True)
                return 0

            lax.fori_loop(0, CH, acc_body, 0)
        plsc.subcore_barrier()
        pltpu.sync_copy(src_hbm.at[wid], idx_v)
        pltpu.sync_copy(w_hbm.at[wid], w_v)

        def nw_body(j, _):
            pltpu.async_copy(deg_sh.at[idx_v.at[j]], deg_v, sem).wait()
            for g in range(B // LANES):
                sl = pl.ds(g * LANES, LANES)
                wv = w_v[j, sl]
                dg = deg_v[sl]
                dg = jnp.where(dg == 0.0, jnp.float32(1.0), dg)
                nw_v[j, sl] = wv / dg
            return 0

        lax.fori_loop(0, CH, nw_body, 0)
        pltpu.sync_copy(nw_v, nw_hbm.at[wid])

    return pl.kernel(
        body,
        out_type=jax.ShapeDtypeStruct((NW, CH, B), jnp.float32),
        mesh=_sc_mesh(),
        scratch_types=[
            pltpu.VMEM((CH, B), jnp.int32),
            pltpu.VMEM((CH, B), jnp.float32),
            pltpu.VMEM((B,), jnp.float32),
            pltpu.VMEM((CH, B), jnp.float32),
            pltpu.VMEM((PT,), jnp.float32),
            pltpu.VMEM_SHARED((NPAD,), jnp.float32),
            pltpu.SemaphoreType.DMA,
        ],
    )


def _make_layer(CH, NPAD, D):
    PT = NPAD // NS

    def body(x_hbm, src_hbm, dst_hbm, nw_hbm, out_hbm,
             isrc_v, idst_v, nw_v, rows_v, agg_sh, sem):
        c = lax.axis_index("c")
        s = lax.axis_index("s")
        wid = s * NC + c

        def zf(r, _):
            for g in range(D // LANES):
                rows_v[r, pl.ds(g * LANES, LANES)] = jnp.zeros((LANES,), jnp.float32)
            return 0

        lax.fori_loop(0, B, zf, 0)
        for t in range(PT // B):
            pltpu.sync_copy(rows_v, agg_sh.at[pl.ds(s * PT + t * B, B)])
        plsc.subcore_barrier()
        pltpu.sync_copy(src_hbm.at[wid], isrc_v)
        pltpu.sync_copy(dst_hbm.at[wid], idst_v)
        pltpu.sync_copy(nw_hbm.at[wid], nw_v)

        def chunk(j, _):
            pltpu.async_copy(x_hbm.at[isrc_v.at[j]], rows_v, sem).wait()

            def scale(gi, _):
                nw16 = nw_v[j, pl.ds(gi * LANES, LANES)]
                for el in range(LANES):
                    e = gi * LANES + el
                    bc16 = jnp.full((LANES,), nw16[el], jnp.float32)
                    for g in range(D // LANES):
                        sl = pl.ds(g * LANES, LANES)
                        rows_v[e, sl] = rows_v[e, sl] * bc16
                return 0

            lax.fori_loop(0, B // LANES, scale, 0)
            pltpu.sync_copy(rows_v, agg_sh.at[idst_v.at[j]], add=True)
            return 0

        lax.fori_loop(0, CH, chunk, 0)
        plsc.subcore_barrier()
        pltpu.sync_copy(agg_sh.at[pl.ds(s * PT, PT)],
                        out_hbm.at[c, pl.ds(s * PT, PT)])

    return pl.kernel(
        body,
        out_type=jax.ShapeDtypeStruct((NC, NPAD, D), jnp.float32),
        mesh=_sc_mesh(),
        scratch_types=[
            pltpu.VMEM((CH, B), jnp.int32),
            pltpu.VMEM((CH, B), jnp.int32),
            pltpu.VMEM((CH, B), jnp.float32),
            pltpu.VMEM((B, D), jnp.float32),
            pltpu.VMEM_SHARED((NPAD, D), jnp.float32),
            pltpu.SemaphoreType.DMA,
        ],
    )


def _mm_relu(part, W, b, Nn, BLK=1000):
    D = W.shape[0]
    H = W.shape[1]

    def body(p_ref, w_ref, b_ref, o_ref):
        acc = p_ref[0] + p_ref[1]
        o_ref[...] = jnp.maximum(
            jnp.dot(acc, w_ref[...], preferred_element_type=jnp.float32)
            + b_ref[...], 0.0)

    return pl.pallas_call(
        body,
        grid=(Nn // BLK,),
        in_specs=[
            pl.BlockSpec((2, BLK, D), lambda i: (0, i, 0)),
            pl.BlockSpec((D, H), lambda i: (0, 0)),
            pl.BlockSpec((1, H), lambda i: (0, 0)),
        ],
        out_specs=pl.BlockSpec((BLK, H), lambda i: (i, 0)),
        out_shape=jax.ShapeDtypeStruct((Nn, H), jnp.float32),
    )(part, W, b.reshape(1, H))


def _final(part, W, b, Wc, bc, Nn, BLK=1000):
    D = W.shape[0]
    H = W.shape[1]
    wct = Wc.reshape(1, H)

    def body(p_ref, w_ref, b_ref, wc_ref, bc_ref, o_ref, acc_ref):
        i = pl.program_id(0)
        h = jnp.maximum(
            jnp.dot(p_ref[0] + p_ref[1], w_ref[...],
                    preferred_element_type=jnp.float32) + b_ref[...], 0.0)
        psum = jnp.sum(h, axis=0, keepdims=True)

        @pl.when(i == 0)
        def _():
            acc_ref[...] = psum

        @pl.when(i > 0)
        def _():
            acc_ref[...] += psum

        @pl.when(i == pl.num_programs(0) - 1)
        def _():
            hg = acc_ref[...] * jnp.float32(1.0 / Nn)
            z = jnp.sum(hg * wc_ref[...], axis=1, keepdims=True) + bc_ref[0, 0]
            o_ref[...] = 1.0 / (1.0 + jnp.exp(-z))

    return pl.pallas_call(
        body,
        grid=(Nn // BLK,),
        in_specs=[
            pl.BlockSpec((2, BLK, D), lambda i: (0, i, 0)),
            pl.BlockSpec((D, H), lambda i: (0, 0)),
            pl.BlockSpec((1, H), lambda i: (0, 0)),
            pl.BlockSpec((1, H), lambda i: (0, 0)),
            pl.BlockSpec(memory_space=pltpu.SMEM),
        ],
        out_specs=pl.BlockSpec((1, 1), lambda i: (0, 0)),
        out_shape=jax.ShapeDtypeStruct((1, 1), jnp.float32),
        scratch_shapes=[pltpu.VMEM((1, H), jnp.float32)],
    )(part, W, b.reshape(1, H), wct, bc.reshape(1, 1))


def kernel(nfeats, edge_index, edge_weights, W1, b1, W2, b2, W3, b3, Wc, bc):
    Nn, D = nfeats.shape
    E = edge_index.shape[1]
    CH = -(-E // (NW * B))
    EPAD = NW * CH * B
    NPAD = -(-Nn // (NS * LANES)) * (NS * LANES)

    pad = EPAD - E
    src_p = jnp.pad(edge_index[0], (0, pad)).reshape(NW, CH, B)
    dst_p = jnp.pad(edge_index[1], (0, pad)).reshape(NW, CH, B)
    w_p = jnp.pad(edge_weights, (0, pad)).reshape(NW, CH, B)

    nw = _make_deg_nw(CH, NPAD)(src_p, w_p)
    layer = _make_layer(CH, NPAD, D)

    part1 = layer(nfeats, src_p, dst_p, nw)
    h1 = _mm_relu(part1, W1, b1, Nn)
    part2 = layer(h1, src_p, dst_p, nw)
    h2 = _mm_relu(part2, W2, b2, Nn)
    part3 = layer(h2, src_p, dst_p, nw)
    return _final(part3, W3, b3, Wc, bc, Nn)
